# baseline (speedup 1.0000x reference)
import functools

import jax
import jax.numpy as jnp
from jax import lax
from jax.experimental import pallas as pl
from jax.experimental.pallas import tpu as pltpu
from jax.experimental.pallas import tpu_sc as plsc

_K = 8192
_D = 256
_N = 8192
_TM = 256
_COMMITMENT_COST = 0.25


def _argmax_onehot_body(x_ref, w_ref, idx_ref, oh_ref, wn_ref):
    @pl.when(pl.program_id(0) == 0)
    def _():
        w = w_ref[...]
        n = jnp.sqrt(jnp.sum(w * w, axis=-1, keepdims=True))
        wn_ref[...] = w / jnp.clip(n, 1e-12, None)

    x = x_ref[...]
    n = jnp.sqrt(jnp.sum(x * x, axis=-1, keepdims=True))
    xn = x / jnp.clip(n, 1e-12, None)
    dist = lax.dot_general(
        xn, wn_ref[...], (((1,), (1,)), ((), ())),
        preferred_element_type=jnp.float32)
    m = jnp.max(dist, axis=1, keepdims=True)
    idx_ref[...] = m.astype(jnp.int32).reshape(1, 1, m.shape[0])
    oh_ref[...] = dist


def _argmax_onehot(x_flat, w):
    n, d = x_flat.shape
    k = w.shape[0]
    g = n // _TM
    idx3, onehot = pl.pallas_call(
        _argmax_onehot_body,
        grid=(g,),
        in_specs=[
            pl.BlockSpec((_TM, d), lambda i: (i, 0)),
            pl.BlockSpec((k, d), lambda i: (0, 0)),
        ],
        out_specs=[
            pl.BlockSpec((1, 1, _TM), lambda i: (i, 0, 0)),
            pl.BlockSpec((_TM, k), lambda i: (i, 0)),
        ],
        out_shape=[
            jax.ShapeDtypeStruct((g, 1, _TM), jnp.int32),
            jax.ShapeDtypeStruct((n, k), jnp.float32),
        ],
        scratch_shapes=[pltpu.VMEM((k, d), jnp.float32)],
    )(x_flat, w)
    return idx3.reshape(n), onehot


@functools.lru_cache(maxsize=None)
def _make_sc_gather(n, d):
    info = plsc.get_sparse_core_info()
    nw = info.num_cores * info.num_subcores
    bpw = n // nw
    mesh = plsc.VectorSubcoreMesh(core_axis_name="c", subcore_axis_name="s")

    @functools.partial(
        pl.kernel,
        out_type=jax.ShapeDtypeStruct((n, d), jnp.float32),
        mesh=mesh,
        scratch_types=[
            pltpu.VMEM((bpw,), jnp.int32),
            pltpu.VMEM((bpw, d), jnp.float32),
            pltpu.SemaphoreType.DMA,
        ],
    )
    def gather_kernel(table_hbm, idx_hbm, out_hbm, idx_v, rows_v, sem):
        wid = lax.axis_index("s") * info.num_cores + lax.axis_index("c")
        base = wid * bpw
        pltpu.sync_copy(idx_hbm.at[pl.ds(base, bpw)], idx_v)
        pltpu.async_copy(table_hbm.at[idx_v], rows_v, sem).wait()
        pltpu.sync_copy(rows_v, out_hbm.at[pl.ds(base, bpw)])

    return gather_kernel


def _loss_body(x_ref, q_ref, idx_ref, sc_ref):
    diff = q_ref[...] - x_ref[...]
    m = jnp.sum(diff * diff) * (1.0 / (_N * _D))
    idx = idx_ref[...]
    eq = (idx[:, None, :] == idx[None, :, :]).astype(jnp.float32)
    c = jnp.sum(eq, axis=0)
    s = jnp.sum(jnp.log(c * 0.125 + 1e-10)) * 0.125
    sc_ref[0] = m
    sc_ref[1] = _COMMITMENT_COST * m
    sc_ref[2] = jnp.exp(-s)


def _losses(x_flat, quant, idx_bt):
    return pl.pallas_call(
        _loss_body,
        out_specs=pl.BlockSpec(memory_space=pltpu.MemorySpace.SMEM),
        out_shape=jax.ShapeDtypeStruct((4,), jnp.float32),
    )(x_flat, quant, idx_bt)


def kernel(inputs, weight):
    b, t, d = inputs.shape
    k = weight.shape[0]
    x_flat = inputs.reshape(b * t, d)
    idx_flat, onehot = _argmax_onehot(x_flat, weight)
    quant = _make_sc_gather(b * t, d)(weight, idx_flat)
    scalars = _losses(x_flat, quant, idx_flat.reshape(b, t))
    quantized_st = quant.reshape(b, t, d)
    encoding_indices_out = idx_flat.reshape(b, t, 1)
    min_encodings = onehot.reshape(b, t, k)
    return (quantized_st, encoding_indices_out, scalars[0], scalars[1],
            scalars[2], min_encodings)


# speedup vs baseline: 2.7243x; 2.7243x over previous
import functools

import jax
import jax.numpy as jnp
from jax import lax
from jax.experimental import pallas as pl
from jax.experimental.pallas import tpu as pltpu
from jax.experimental.pallas import tpu_sc as plsc

_K = 8192
_D = 256
_B = 8
_T = 1024
_N = _B * _T
_TM = 256
_COMMITMENT_COST = 0.25


def _vq_body(x_ref, w_ref, idx_ref, oh_ref, sc_ref,
             wn_ref, cv_ref, idxall_ref, acc_ref):
    i = pl.program_id(0)

    @pl.when(i == 0)
    def _():
        w = w_ref[...]
        nw = jnp.sqrt(jnp.sum(w * w, axis=-1, keepdims=True))
        wn_ref[...] = w / jnp.clip(nw, 1e-12, None)
        lane = lax.broadcasted_iota(jnp.int32, cv_ref.shape, 1)
        cv_ref[...] = jnp.where(lane == 0, nw, 0.0).astype(jnp.bfloat16)
        acc_ref[0] = 0.0

    x = x_ref[...]
    s = jnp.sum(x * x, axis=-1, keepdims=True)
    n = jnp.sqrt(s)
    xn = x / jnp.clip(n, 1e-12, None)
    dist = lax.dot_general(
        xn, wn_ref[...], (((1,), (1,)), ((), ())),
        preferred_element_type=jnp.float32)
    kdim = dist.shape[1]
    m = jnp.max(dist, axis=1, keepdims=True)
    colf = lax.broadcasted_iota(jnp.int32, dist.shape, 1).astype(jnp.float32)
    vf = jnp.where(dist == m, colf, float(kdim))
    idxf = jnp.min(vf, axis=1, keepdims=True)
    idx = idxf.astype(jnp.int32)
    idx_ref[...] = idx.reshape(1, 1, idx.shape[0])
    oh_ref[...] = (vf == idxf).astype(jnp.float32)

    z = lax.dot_general(oh_ref[...].astype(jnp.bfloat16), cv_ref[...],
                        (((1,), (0,)), ((), ())),
                        preferred_element_type=jnp.float32)
    w1 = z[:, 0:1]
    acc_ref[0] += jnp.sum(w1 * w1 - 2.0 * (m * n) * w1 + s)

    tiles_per_b = _T // _TM
    b = i // tiles_per_b
    tcol = (i % tiles_per_b) * _TM
    idxall_ref[pl.ds(b, 1), pl.ds(tcol, _TM)] = idx.reshape(1, _TM)

    @pl.when(i == pl.num_programs(0) - 1)
    def _():
        ia = idxall_ref[...]
        eq = (ia[:, None, :] == ia[None, :, :]).astype(jnp.float32)
        c = jnp.sum(eq, axis=0)
        ent = jnp.sum(jnp.log(c * (1.0 / _B) + 1e-10)) * (1.0 / _B)
        msum = acc_ref[0] * (1.0 / (_N * _D))
        sc_ref[0] = msum
        sc_ref[1] = _COMMITMENT_COST * msum
        sc_ref[2] = jnp.exp(-ent)


def _vq_main(x_flat, w):
    n, d = x_flat.shape
    k = w.shape[0]
    g = n // _TM
    idx3, onehot, scalars = pl.pallas_call(
        _vq_body,
        grid=(g,),
        in_specs=[
            pl.BlockSpec((_TM, d), lambda i: (i, 0)),
            pl.BlockSpec((k, d), lambda i: (0, 0)),
        ],
        out_specs=[
            pl.BlockSpec((1, 1, _TM), lambda i: (i, 0, 0)),
            pl.BlockSpec((_TM, k), lambda i: (i, 0)),
            pl.BlockSpec(memory_space=pltpu.MemorySpace.SMEM),
        ],
        out_shape=[
            jax.ShapeDtypeStruct((g, 1, _TM), jnp.int32),
            jax.ShapeDtypeStruct((n, k), jnp.float32),
            jax.ShapeDtypeStruct((4,), jnp.float32),
        ],
        scratch_shapes=[
            pltpu.VMEM((k, d), jnp.float32),
            pltpu.VMEM((k, 8), jnp.bfloat16),
            pltpu.VMEM((_B, _T), jnp.int32),
            pltpu.SMEM((1,), jnp.float32),
        ],
    )(x_flat, w)
    return idx3.reshape(n), onehot, scalars


@functools.lru_cache(maxsize=None)
def _make_sc_gather(n, d):
    info = plsc.get_sparse_core_info()
    nw = info.num_cores * info.num_subcores
    bpw = n // nw
    mesh = plsc.VectorSubcoreMesh(core_axis_name="c", subcore_axis_name="s")

    @functools.partial(
        pl.kernel,
        out_type=jax.ShapeDtypeStruct((n, d), jnp.float32),
        mesh=mesh,
        scratch_types=[
            pltpu.VMEM((bpw,), jnp.int32),
            pltpu.VMEM((bpw, d), jnp.float32),
            pltpu.SemaphoreType.DMA,
        ],
    )
    def gather_kernel(table_hbm, idx_hbm, out_hbm, idx_v, rows_v, sem):
        wid = lax.axis_index("s") * info.num_cores + lax.axis_index("c")
        base = wid * bpw
        pltpu.sync_copy(idx_hbm.at[pl.ds(base, bpw)], idx_v)
        pltpu.async_copy(table_hbm.at[idx_v], rows_v, sem).wait()
        pltpu.sync_copy(rows_v, out_hbm.at[pl.ds(base, bpw)])

    return gather_kernel


def kernel(inputs, weight):
    b, t, d = inputs.shape
    k = weight.shape[0]
    x_flat = inputs.reshape(b * t, d)
    idx_flat, onehot, scalars = _vq_main(x_flat, weight)
    quant = _make_sc_gather(b * t, d)(weight, idx_flat)
    quantized_st = quant.reshape(b, t, d)
    encoding_indices_out = idx_flat.reshape(b, t, 1)
    min_encodings = onehot.reshape(b, t, k)
    return (quantized_st, encoding_indices_out, scalars[0], scalars[1],
            scalars[2], min_encodings)


# speedup vs baseline: 3.0195x; 1.1083x over previous
import functools

import jax
import jax.numpy as jnp
from jax import lax
from jax.experimental import pallas as pl
from jax.experimental.pallas import tpu as pltpu
from jax.experimental.pallas import tpu_sc as plsc

_K = 8192
_D = 256
_N = 8192
_TM = 256
_COMMITMENT_COST = 0.25


def _argmax_onehot_body(x_ref, w_ref, idx_ref, oh_ref, wn_ref):
    @pl.when(pl.program_id(0) == 0)
    def _():
        w = w_ref[...]
        n = jnp.sqrt(jnp.sum(w * w, axis=-1, keepdims=True))
        wn_ref[...] = w / jnp.clip(n, 1e-12, None)

    x = x_ref[...]
    n = jnp.sqrt(jnp.sum(x * x, axis=-1, keepdims=True))
    xn = x / jnp.clip(n, 1e-12, None)
    dist = lax.dot_general(
        xn, wn_ref[...], (((1,), (1,)), ((), ())),
        preferred_element_type=jnp.float32)
    kdim = dist.shape[1]
    m = jnp.max(dist, axis=1, keepdims=True)
    colf = lax.broadcasted_iota(jnp.int32, dist.shape, 1).astype(jnp.float32)
    vf = jnp.where(dist == m, colf, float(kdim))
    idxf = jnp.min(vf, axis=1, keepdims=True)
    idx_ref[...] = idxf.astype(jnp.int32).reshape(1, 1, idxf.shape[0])
    oh_ref[...] = (vf == idxf).astype(jnp.float32)


def _argmax_onehot(x_flat, w):
    n, d = x_flat.shape
    k = w.shape[0]
    g = n // _TM
    idx3, onehot = pl.pallas_call(
        _argmax_onehot_body,
        grid=(g,),
        in_specs=[
            pl.BlockSpec((_TM, d), lambda i: (i, 0)),
            pl.BlockSpec((k, d), lambda i: (0, 0)),
        ],
        out_specs=[
            pl.BlockSpec((1, 1, _TM), lambda i: (i, 0, 0)),
            pl.BlockSpec((_TM, k), lambda i: (i, 0)),
        ],
        out_shape=[
            jax.ShapeDtypeStruct((g, 1, _TM), jnp.int32),
            jax.ShapeDtypeStruct((n, k), jnp.float32),
        ],
        scratch_shapes=[pltpu.VMEM((k, d), jnp.float32)],
    )(x_flat, w)
    return idx3.reshape(n), onehot


@functools.lru_cache(maxsize=None)
def _make_sc_gather(n, d):
    info = plsc.get_sparse_core_info()
    nw = info.num_cores * info.num_subcores
    bpw = n // nw
    half = bpw // 2
    mesh = plsc.VectorSubcoreMesh(core_axis_name="c", subcore_axis_name="s")

    @functools.partial(
        pl.kernel,
        out_type=jax.ShapeDtypeStruct((n, d), jnp.float32),
        mesh=mesh,
        scratch_types=[
            pltpu.VMEM((bpw,), jnp.int32),
            pltpu.VMEM((half, d), jnp.float32),
            pltpu.VMEM((half, d), jnp.float32),
            pltpu.SemaphoreType.DMA,
            pltpu.SemaphoreType.DMA,
            pltpu.SemaphoreType.DMA,
            pltpu.SemaphoreType.DMA,
        ],
    )
    def gather_kernel(table_hbm, idx_hbm, out_hbm,
                      idx_v, rows_a, rows_b, sem_a, sem_b, sem_wa, sem_wb):
        wid = lax.axis_index("s") * info.num_cores + lax.axis_index("c")
        base = wid * bpw
        pltpu.sync_copy(idx_hbm.at[pl.ds(base, bpw)], idx_v)
        ga = pltpu.async_copy(table_hbm.at[idx_v.at[pl.ds(0, half)]],
                              rows_a, sem_a)
        gb = pltpu.async_copy(table_hbm.at[idx_v.at[pl.ds(half, half)]],
                              rows_b, sem_b)
        ga.wait()
        wa = pltpu.async_copy(rows_a, out_hbm.at[pl.ds(base, half)], sem_wa)
        gb.wait()
        wb = pltpu.async_copy(rows_b, out_hbm.at[pl.ds(base + half, half)],
                              sem_wb)
        wa.wait()
        wb.wait()

    return gather_kernel


def _loss_body(x_ref, q_ref, idx_ref, sc_ref):
    diff = q_ref[...] - x_ref[...]
    m = jnp.sum(diff * diff) * (1.0 / (_N * _D))
    idx = idx_ref[...]
    eq = (idx[:, None, :] == idx[None, :, :]).astype(jnp.float32)
    c = jnp.sum(eq, axis=0)
    s = jnp.sum(jnp.log(c * 0.125 + 1e-10)) * 0.125
    sc_ref[0] = m
    sc_ref[1] = _COMMITMENT_COST * m
    sc_ref[2] = jnp.exp(-s)


def _losses(x_flat, quant, idx_bt):
    return pl.pallas_call(
        _loss_body,
        out_specs=pl.BlockSpec(memory_space=pltpu.MemorySpace.SMEM),
        out_shape=jax.ShapeDtypeStruct((4,), jnp.float32),
    )(x_flat, quant, idx_bt)


def kernel(inputs, weight):
    b, t, d = inputs.shape
    k = weight.shape[0]
    x_flat = inputs.reshape(b * t, d)
    idx_flat, onehot = _argmax_onehot(x_flat, weight)
    quant = _make_sc_gather(b * t, d)(weight, idx_flat)
    scalars = _losses(x_flat, quant, idx_flat.reshape(b, t))
    quantized_st = quant.reshape(b, t, d)
    encoding_indices_out = idx_flat.reshape(b, t, 1)
    min_encodings = onehot.reshape(b, t, k)
    return (quantized_st, encoding_indices_out, scalars[0], scalars[1],
            scalars[2], min_encodings)
